# Initial kernel scaffold; baseline (speedup 1.0000x reference)
import jax
import jax.numpy as jnp
from jax.experimental import pallas as pl


def kernel(h, e, edge_index, W1, b1, W2, b2, nn_bias, gamma, beta):
    raise NotImplementedError("write your pallas kernel here")


# speedup vs baseline: 3.8759x; 3.8759x over previous
import functools

import jax
import jax.numpy as jnp
from jax import lax
from jax.experimental import pallas as pl
from jax.experimental.pallas import tpu as pltpu
from jax.experimental.pallas import tpu_sc as plsc

N_NODES = 10000
N_EDGES = 160000
IN_DIM = 16
OUT_DIM = 16
E_DIM = 6
EDGE_H = 64
AUG = 32

NW = 32
EDGES_PER_W = N_EDGES // NW
CH = 125
NCH = EDGES_PER_W // CH
N_PAD = 10240
ROWS_PER_TILE = N_PAD // 16

BE = 2000
GRID = N_EDGES // BE

@functools.cache
def _sc_mesh():
    return plsc.VectorSubcoreMesh(core_axis_name="c", subcore_axis_name="s",
                                  num_cores=2)


@functools.cache
def _gather_hsrc_kernel():
    @functools.partial(
        pl.kernel,
        mesh=_sc_mesh(),
        compiler_params=pltpu.CompilerParams(use_tc_tiling_on_sc=False),
        out_type=jax.ShapeDtypeStruct((NW, NCH, CH, IN_DIM), jnp.float32),
        scratch_types=[
            pltpu.VMEM((NCH, CH), jnp.int32),
            pltpu.VMEM((CH, IN_DIM), jnp.float32),
        ],
    )
    def _gather_hsrc(h_hbm, src_hbm, out_hbm, idx_v, rows_v):
        c = lax.axis_index("c")
        s = lax.axis_index("s")
        w = c * 16 + s
        pltpu.sync_copy(src_hbm.at[w], idx_v)

        def body(j, carry):
            pltpu.sync_copy(h_hbm.at[idx_v.at[j]], rows_v)
            pltpu.sync_copy(rows_v, out_hbm.at[w, j])
            return carry

        lax.fori_loop(0, NCH, body, 0)

    return _gather_hsrc


def _gather_hsrc(h, src_rs):
    return _gather_hsrc_kernel()(h, src_rs).reshape(N_EDGES, IN_DIM)


@functools.cache
def _scatter_agg_kernel():
    @functools.partial(
        pl.kernel,
        mesh=_sc_mesh(),
        compiler_params=pltpu.CompilerParams(use_tc_tiling_on_sc=False),
        out_type=jax.ShapeDtypeStruct((2, N_PAD, AUG), jnp.float32),
        scratch_types=[
            pltpu.VMEM((NCH, CH), jnp.int32),
            pltpu.VMEM((CH, AUG), jnp.float32),
            pltpu.VMEM_SHARED((N_PAD, AUG), jnp.float32),
        ],
    )
    def _scatter(msg_hbm, dst_hbm, zero_hbm, out_hbm, idx_v, rows_v, agg_sh):
        c = lax.axis_index("c")
        s = lax.axis_index("s")
        w = c * 16 + s
        pltpu.sync_copy(zero_hbm.at[pl.ds(s * ROWS_PER_TILE, ROWS_PER_TILE)],
                        agg_sh.at[pl.ds(s * ROWS_PER_TILE, ROWS_PER_TILE)])
        plsc.subcore_barrier()
        pltpu.sync_copy(dst_hbm.at[w], idx_v)

        def body(j, carry):
            pltpu.sync_copy(msg_hbm.at[w, j], rows_v)
            pltpu.sync_copy(rows_v, agg_sh.at[idx_v.at[j]], add=True)
            return carry

        lax.fori_loop(0, NCH, body, 0)
        plsc.subcore_barrier()
        pltpu.sync_copy(agg_sh.at[pl.ds(s * ROWS_PER_TILE, ROWS_PER_TILE)],
                        out_hbm.at[c, pl.ds(s * ROWS_PER_TILE, ROWS_PER_TILE)])

    return _scatter


def _scatter_agg(msg, dst_rs, zero):
    return _scatter_agg_kernel()(msg.reshape(NW, NCH, CH, AUG), dst_rs, zero)


def _dense_body(e_ref, hs_ref, w1_ref, b1_ref, w2_ref, b2_ref, r_ref, s_ref,
                out_ref):
    e = e_ref[...]
    hid = jnp.maximum(
        jnp.dot(e, w1_ref[...], preferred_element_type=jnp.float32)
        + b1_ref[...], 0.0)
    ew = (jnp.dot(hid, w2_ref[...], preferred_element_type=jnp.float32)
          + b2_ref[...])
    hrep = jnp.dot(hs_ref[...], r_ref[...], preferred_element_type=jnp.float32)
    msg = jnp.dot(hrep * ew, s_ref[...], preferred_element_type=jnp.float32)
    out_ref[:, 0:OUT_DIM] = msg
    col = lax.broadcasted_iota(jnp.int32, (BE, AUG - OUT_DIM), 1)
    out_ref[:, OUT_DIM:AUG] = jnp.where(col == 0, 1.0, 0.0)


def _dense_msg(e, h_src, W1, b1, W2, b2, R, S):
    return pl.pallas_call(
        _dense_body,
        grid=(GRID,),
        in_specs=[
            pl.BlockSpec((BE, E_DIM), lambda i: (i, 0)),
            pl.BlockSpec((BE, IN_DIM), lambda i: (i, 0)),
            pl.BlockSpec((E_DIM, EDGE_H), lambda i: (0, 0)),
            pl.BlockSpec((1, EDGE_H), lambda i: (0, 0)),
            pl.BlockSpec((EDGE_H, IN_DIM * OUT_DIM), lambda i: (0, 0)),
            pl.BlockSpec((1, IN_DIM * OUT_DIM), lambda i: (0, 0)),
            pl.BlockSpec((IN_DIM, IN_DIM * OUT_DIM), lambda i: (0, 0)),
            pl.BlockSpec((IN_DIM * OUT_DIM, OUT_DIM), lambda i: (0, 0)),
        ],
        out_specs=pl.BlockSpec((BE, AUG), lambda i: (i, 0)),
        out_shape=jax.ShapeDtypeStruct((N_EDGES, AUG), jnp.float32),
    )(e, h_src, W1, b1, W2, b2, R, S)


def _final_body(parts_ref, bias_ref, gamma_ref, beta_ref, out_ref):
    p = parts_ref[0, 0:N_NODES] + parts_ref[1, 0:N_NODES]
    agg = p[:, 0:OUT_DIM]
    deg = p[:, OUT_DIM:OUT_DIM + 1]
    rst = agg / jnp.maximum(deg, 1.0) + bias_ref[...]
    rst = jnp.maximum(rst, 0.0)
    mean = jnp.mean(rst, axis=0, keepdims=True)
    var = jnp.mean((rst - mean) * (rst - mean), axis=0, keepdims=True)
    out_ref[...] = ((rst - mean) * lax.rsqrt(var + 1e-5) * gamma_ref[...]
                    + beta_ref[...])


def _finalize(parts, nn_bias, gamma, beta):
    return pl.pallas_call(
        _final_body,
        in_specs=[
            pl.BlockSpec((2, N_PAD, AUG), lambda: (0, 0, 0)),
            pl.BlockSpec((1, OUT_DIM), lambda: (0, 0)),
            pl.BlockSpec((1, OUT_DIM), lambda: (0, 0)),
            pl.BlockSpec((1, OUT_DIM), lambda: (0, 0)),
        ],
        out_specs=pl.BlockSpec((N_NODES, OUT_DIM), lambda: (0, 0)),
        out_shape=jax.ShapeDtypeStruct((N_NODES, OUT_DIM), jnp.float32),
    )(parts, nn_bias, gamma, beta)


def kernel(h, e, edge_index, W1, b1, W2, b2, nn_bias, gamma, beta):
    src = edge_index[0].reshape(NW, NCH, CH)
    dst = edge_index[1].reshape(NW, NCH, CH)

    h_src = _gather_hsrc(h, src)

    eye = jnp.eye(OUT_DIM, dtype=jnp.float32)
    R = jnp.kron(jnp.eye(IN_DIM, dtype=jnp.float32),
                 jnp.ones((1, OUT_DIM), jnp.float32))
    S = jnp.tile(eye, (IN_DIM, 1))
    msg = _dense_msg(e, h_src, W1, b1.reshape(1, EDGE_H), W2,
                     b2.reshape(1, IN_DIM * OUT_DIM), R, S)

    zero = jnp.zeros((N_PAD, AUG), jnp.float32)
    parts = _scatter_agg(msg, dst, zero)

    return _finalize(parts, nn_bias.reshape(1, OUT_DIM),
                     gamma.reshape(1, OUT_DIM), beta.reshape(1, OUT_DIM))


# speedup vs baseline: 4.1967x; 1.0828x over previous
import functools

import jax
import jax.numpy as jnp
from jax import lax
from jax.experimental import pallas as pl
from jax.experimental.pallas import tpu as pltpu
from jax.experimental.pallas import tpu_sc as plsc

N_NODES = 10000
N_EDGES = 160000
IN_DIM = 16
OUT_DIM = 16
E_DIM = 6
EDGE_H = 64
AUG = 32

NW = 32
EDGES_PER_W = N_EDGES // NW
CH = 125
NCH = EDGES_PER_W // CH
N_PAD = 10240
ROWS_PER_TILE = N_PAD // 16

BE = 2000
GRID = N_EDGES // BE

@functools.cache
def _sc_mesh():
    return plsc.VectorSubcoreMesh(core_axis_name="c", subcore_axis_name="s",
                                  num_cores=2)


G_EDGES = 1000
N_GROUPS = EDGES_PER_W // G_EDGES
CH_PER_G = G_EDGES // CH


@functools.cache
def _gather_hsrc_kernel():
    @functools.partial(
        pl.kernel,
        mesh=_sc_mesh(),
        compiler_params=pltpu.CompilerParams(use_tc_tiling_on_sc=False),
        out_type=jax.ShapeDtypeStruct((N_EDGES, IN_DIM), jnp.float32),
        scratch_types=[
            pltpu.VMEM((NCH, CH), jnp.int32),
            pltpu.VMEM((2, G_EDGES, IN_DIM), jnp.float32),
            pltpu.SemaphoreType.DMA,
            pltpu.SemaphoreType.DMA,
        ],
    )
    def _gather_hsrc(h_hbm, src_hbm, out_hbm, idx_v, rows_v, sem_g, sem_w):
        c = lax.axis_index("c")
        s = lax.axis_index("s")
        w = c * 16 + s
        base = w * EDGES_PER_W
        pltpu.sync_copy(src_hbm.at[w], idx_v)

        def fire(g, b):
            for k in range(CH_PER_G):
                pltpu.async_copy(
                    h_hbm.at[idx_v.at[g * CH_PER_G + k]],
                    rows_v.at[b, pl.ds(k * CH, CH)], sem_g)

        def drain(g, b):
            for k in range(CH_PER_G):
                pltpu.make_async_copy(
                    h_hbm.at[idx_v.at[g * CH_PER_G + k]],
                    rows_v.at[b, pl.ds(k * CH, CH)], sem_g).wait()

        fire(0, 0)
        for g in range(N_GROUPS):
            b = g % 2
            drain(g, b)
            if g >= 2:
                pltpu.make_async_copy(
                    rows_v.at[b],
                    out_hbm.at[pl.ds(base + (g - 2) * G_EDGES, G_EDGES)],
                    sem_w).wait()
            if g + 1 < N_GROUPS:
                fire(g + 1, (g + 1) % 2)
            pltpu.async_copy(
                rows_v.at[b],
                out_hbm.at[pl.ds(base + g * G_EDGES, G_EDGES)], sem_w)
        for g in (N_GROUPS - 2, N_GROUPS - 1):
            pltpu.make_async_copy(
                rows_v.at[g % 2],
                out_hbm.at[pl.ds(base + g * G_EDGES, G_EDGES)], sem_w).wait()

    return _gather_hsrc


def _gather_hsrc(h, src_rs):
    return _gather_hsrc_kernel()(h, src_rs)


@functools.cache
def _scatter_agg_kernel():
    @functools.partial(
        pl.kernel,
        mesh=_sc_mesh(),
        compiler_params=pltpu.CompilerParams(use_tc_tiling_on_sc=False),
        out_type=jax.ShapeDtypeStruct((2, N_PAD, AUG), jnp.float32),
        scratch_types=[
            pltpu.VMEM((NCH, CH), jnp.int32),
            pltpu.VMEM((2, G_EDGES, AUG), jnp.float32),
            pltpu.VMEM_SHARED((N_PAD, AUG), jnp.float32),
            pltpu.SemaphoreType.DMA,
            pltpu.SemaphoreType.DMA,
        ],
    )
    def _scatter(msg_hbm, dst_hbm, zero_hbm, out_hbm, idx_v, rows_v, agg_sh,
                 sem_r, sem_s):
        c = lax.axis_index("c")
        s = lax.axis_index("s")
        w = c * 16 + s
        base = w * EDGES_PER_W
        pltpu.sync_copy(zero_hbm.at[pl.ds(s * ROWS_PER_TILE, ROWS_PER_TILE)],
                        agg_sh.at[pl.ds(s * ROWS_PER_TILE, ROWS_PER_TILE)])
        pltpu.sync_copy(dst_hbm.at[w], idx_v)
        plsc.subcore_barrier()

        def read(g, b):
            pltpu.async_copy(
                msg_hbm.at[pl.ds(base + g * G_EDGES, G_EDGES)],
                rows_v.at[b], sem_r)

        def wait_read(g, b):
            pltpu.make_async_copy(
                msg_hbm.at[pl.ds(base + g * G_EDGES, G_EDGES)],
                rows_v.at[b], sem_r).wait()

        def scatters(g, b, do_wait):
            for k in range(CH_PER_G):
                a = (rows_v.at[b, pl.ds(k * CH, CH)],
                     agg_sh.at[idx_v.at[g * CH_PER_G + k]])
                if do_wait:
                    pltpu.make_async_copy(*a, sem_s).wait()
                else:
                    pltpu.async_copy(*a, sem_s, add=True)

        read(0, 0)
        for g in range(N_GROUPS):
            b = g % 2
            wait_read(g, b)
            if g >= 2:
                scatters(g - 2, b, True)
            if g + 1 < N_GROUPS:
                read(g + 1, (g + 1) % 2)
            scatters(g, b, False)
        for g in (N_GROUPS - 2, N_GROUPS - 1):
            scatters(g, g % 2, True)
        plsc.subcore_barrier()
        pltpu.sync_copy(agg_sh.at[pl.ds(s * ROWS_PER_TILE, ROWS_PER_TILE)],
                        out_hbm.at[c, pl.ds(s * ROWS_PER_TILE, ROWS_PER_TILE)])

    return _scatter


def _scatter_agg(msg, dst_rs, zero):
    return _scatter_agg_kernel()(msg, dst_rs, zero)


def _dense_body(e_ref, hs_ref, w1_ref, b1_ref, w2_ref, b2_ref, r_ref, s_ref,
                out_ref):
    e = e_ref[...]
    hid = jnp.maximum(
        jnp.dot(e, w1_ref[...], preferred_element_type=jnp.float32)
        + b1_ref[...], 0.0)
    ew = (jnp.dot(hid, w2_ref[...], preferred_element_type=jnp.float32)
          + b2_ref[...])
    hrep = jnp.dot(hs_ref[...], r_ref[...], preferred_element_type=jnp.float32)
    msg = jnp.dot(hrep * ew, s_ref[...], preferred_element_type=jnp.float32)
    out_ref[:, 0:OUT_DIM] = msg
    col = lax.broadcasted_iota(jnp.int32, (BE, AUG - OUT_DIM), 1)
    out_ref[:, OUT_DIM:AUG] = jnp.where(col == 0, 1.0, 0.0)


def _dense_msg(e, h_src, W1, b1, W2, b2, R, S):
    return pl.pallas_call(
        _dense_body,
        grid=(GRID,),
        in_specs=[
            pl.BlockSpec((BE, E_DIM), lambda i: (i, 0)),
            pl.BlockSpec((BE, IN_DIM), lambda i: (i, 0)),
            pl.BlockSpec((E_DIM, EDGE_H), lambda i: (0, 0)),
            pl.BlockSpec((1, EDGE_H), lambda i: (0, 0)),
            pl.BlockSpec((EDGE_H, IN_DIM * OUT_DIM), lambda i: (0, 0)),
            pl.BlockSpec((1, IN_DIM * OUT_DIM), lambda i: (0, 0)),
            pl.BlockSpec((IN_DIM, IN_DIM * OUT_DIM), lambda i: (0, 0)),
            pl.BlockSpec((IN_DIM * OUT_DIM, OUT_DIM), lambda i: (0, 0)),
        ],
        out_specs=pl.BlockSpec((BE, AUG), lambda i: (i, 0)),
        out_shape=jax.ShapeDtypeStruct((N_EDGES, AUG), jnp.float32),
    )(e, h_src, W1, b1, W2, b2, R, S)


def _final_body(parts_ref, bias_ref, gamma_ref, beta_ref, out_ref):
    p = parts_ref[0, 0:N_NODES] + parts_ref[1, 0:N_NODES]
    agg = p[:, 0:OUT_DIM]
    deg = p[:, OUT_DIM:OUT_DIM + 1]
    rst = agg / jnp.maximum(deg, 1.0) + bias_ref[...]
    rst = jnp.maximum(rst, 0.0)
    mean = jnp.mean(rst, axis=0, keepdims=True)
    var = jnp.mean((rst - mean) * (rst - mean), axis=0, keepdims=True)
    out_ref[...] = ((rst - mean) * lax.rsqrt(var + 1e-5) * gamma_ref[...]
                    + beta_ref[...])


def _finalize(parts, nn_bias, gamma, beta):
    return pl.pallas_call(
        _final_body,
        in_specs=[
            pl.BlockSpec((2, N_PAD, AUG), lambda: (0, 0, 0)),
            pl.BlockSpec((1, OUT_DIM), lambda: (0, 0)),
            pl.BlockSpec((1, OUT_DIM), lambda: (0, 0)),
            pl.BlockSpec((1, OUT_DIM), lambda: (0, 0)),
        ],
        out_specs=pl.BlockSpec((N_NODES, OUT_DIM), lambda: (0, 0)),
        out_shape=jax.ShapeDtypeStruct((N_NODES, OUT_DIM), jnp.float32),
    )(parts, nn_bias, gamma, beta)


def kernel(h, e, edge_index, W1, b1, W2, b2, nn_bias, gamma, beta):
    src = edge_index[0].reshape(NW, NCH, CH)
    dst = edge_index[1].reshape(NW, NCH, CH)

    h_src = _gather_hsrc(h, src)

    eye = jnp.eye(OUT_DIM, dtype=jnp.float32)
    R = jnp.kron(jnp.eye(IN_DIM, dtype=jnp.float32),
                 jnp.ones((1, OUT_DIM), jnp.float32))
    S = jnp.tile(eye, (IN_DIM, 1))
    msg = _dense_msg(e, h_src, W1, b1.reshape(1, EDGE_H), W2,
                     b2.reshape(1, IN_DIM * OUT_DIM), R, S)

    zero = jnp.zeros((N_PAD, AUG), jnp.float32)
    parts = _scatter_agg(msg, dst, zero)

    return _finalize(parts, nn_bias.reshape(1, OUT_DIM),
                     gamma.reshape(1, OUT_DIM), beta.reshape(1, OUT_DIM))
